# Initial kernel scaffold; baseline (speedup 1.0000x reference)
import jax
import jax.numpy as jnp
from jax.experimental import pallas as pl


def kernel(x, edge_index, W0, b0, W1, b1):
    raise NotImplementedError("write your pallas kernel here")


# speedup vs baseline: 8.1277x; 8.1277x over previous
import functools

import jax
import jax.numpy as jnp
from jax import lax
from jax.experimental import pallas as pl
from jax.experimental.pallas import tpu as pltpu
from jax.experimental.pallas import tpu_sc as plsc

_N = 10000
_E = 320000
_D = 128
_NC = 2
_NS = 16
_NW = _NC * _NS
_CH = 128
_CPW = 80
_EP = _NW * _CPW * _CH
_NP = 10240
_RPT = _NP // _NS
_NPAD = 64


_MESH = dict(core_axis_name="c", subcore_axis_name="s")


def _worker_id():
    return lax.axis_index("s") * _NC + lax.axis_index("c")


def _zero_fill(buf, rows, width):
    def row(i, carry):
        def col(k, c):
            buf[i, pl.ds(k * 16, 16)] = jnp.zeros((16,), jnp.float32)
            return c
        return lax.fori_loop(0, width // 16, col, carry)
    lax.fori_loop(0, rows, row, 0)


@functools.partial(
    pl.kernel,
    out_type=jax.ShapeDtypeStruct((_NC, _NP, _D), jnp.float32),
    mesh=plsc.VectorSubcoreMesh(**_MESH),
    scratch_types=[
        pltpu.VMEM((_CPW, _CH), jnp.int32),
        pltpu.VMEM((_CPW, _CH), jnp.int32),
        pltpu.VMEM((_CH, _D), jnp.float32),
        pltpu.VMEM_SHARED((_NP, _D), jnp.float32),
    ],
)
def _sc_propagate(g_hbm, src_hbm, dst_hbm, out_hbm, src_v, dst_v, buf, acc):
    cid = lax.axis_index("c")
    sid = lax.axis_index("s")
    wid = _worker_id()
    pltpu.sync_copy(src_hbm.at[wid], src_v)
    pltpu.sync_copy(dst_hbm.at[wid], dst_v)
    _zero_fill(buf, _CH, _D)
    def zstripe(i, c):
        pltpu.sync_copy(buf, acc.at[pl.ds(sid * _RPT + i * _CH, _CH)])
        return c
    lax.fori_loop(0, _RPT // _CH, zstripe, 0)
    plsc.subcore_barrier()
    def step(j, c):
        pltpu.sync_copy(g_hbm.at[src_v.at[j]], buf)
        pltpu.sync_copy(buf, acc.at[dst_v.at[j]], add=True)
        return c
    lax.fori_loop(0, _CPW, step, 0)
    plsc.subcore_barrier()
    def wb(i, c):
        sl = pl.ds(sid * _RPT + i * _CH, _CH)
        pltpu.sync_copy(acc.at[sl], out_hbm.at[cid].at[sl])
        return c
    lax.fori_loop(0, _RPT // _CH, wb, 0)


@functools.partial(
    pl.kernel,
    out_type=jax.ShapeDtypeStruct((_NC, _NP, 16), jnp.float32),
    mesh=plsc.VectorSubcoreMesh(**_MESH),
    scratch_types=[
        pltpu.VMEM((_CPW, _CH), jnp.int32),
        pltpu.VMEM((_CH, 16), jnp.float32),
        pltpu.VMEM((_CH, 16), jnp.float32),
        pltpu.VMEM_SHARED((_NP, 16), jnp.float32),
    ],
)
def _sc_degree(dst_hbm, out_hbm, dst_v, ones_v, zeros_v, acc):
    cid = lax.axis_index("c")
    sid = lax.axis_index("s")
    wid = _worker_id()
    pltpu.sync_copy(dst_hbm.at[wid], dst_v)
    _zero_fill(zeros_v, _CH, 16)
    def fill1(i, c):
        ones_v[i, pl.ds(0, 16)] = jnp.ones((16,), jnp.float32)
        return c
    lax.fori_loop(0, _CH, fill1, 0)
    def zstripe(i, c):
        pltpu.sync_copy(zeros_v, acc.at[pl.ds(sid * _RPT + i * _CH, _CH)])
        return c
    lax.fori_loop(0, _RPT // _CH, zstripe, 0)
    plsc.subcore_barrier()
    def step(j, c):
        pltpu.sync_copy(ones_v, acc.at[dst_v.at[j]], add=True)
        return c
    lax.fori_loop(0, _CPW, step, 0)
    plsc.subcore_barrier()
    def wb(i, c):
        sl = pl.ds(sid * _RPT + i * _CH, _CH)
        pltpu.sync_copy(acc.at[sl], out_hbm.at[cid].at[sl])
        return c
    lax.fori_loop(0, _RPT // _CH, wb, 0)


_R = 1024


def _norm_block(dp):
    deg = dp[0, :, 0:1] + dp[1, :, 0:1]
    return lax.rsqrt(jnp.maximum(deg, 1.0))


def _tc_scale_body(dp_ref, x_ref, g_ref):
    g_ref[...] = x_ref[...] * _norm_block(dp_ref[...])


def _tc_scale(dp, xp):
    return pl.pallas_call(
        _tc_scale_body,
        grid=(_NP // _R,),
        in_specs=[
            pl.BlockSpec((_NC, _R, 16), lambda i: (0, i, 0)),
            pl.BlockSpec((_R, _D), lambda i: (i, 0)),
        ],
        out_specs=pl.BlockSpec((_R, _D), lambda i: (i, 0)),
        out_shape=jax.ShapeDtypeStruct((_NP, _D), jnp.float32),
    )(dp, xp)


def _tc_combine_body(dp_ref, p_ref, h_ref, g_ref):
    norm = _norm_block(dp_ref[...])
    h = (p_ref[0] + p_ref[1]) * norm
    h_ref[...] = h
    g_ref[...] = h * norm


def _tc_combine(dp, p):
    return pl.pallas_call(
        _tc_combine_body,
        grid=(_NP // _R,),
        in_specs=[
            pl.BlockSpec((_NC, _R, 16), lambda i: (0, i, 0)),
            pl.BlockSpec((_NC, _R, _D), lambda i: (0, i, 0)),
        ],
        out_specs=[
            pl.BlockSpec((_R, _D), lambda i: (i, 0)),
            pl.BlockSpec((_R, _D), lambda i: (i, 0)),
        ],
        out_shape=[
            jax.ShapeDtypeStruct((_NP, _D), jnp.float32),
            jax.ShapeDtypeStruct((_NP, _D), jnp.float32),
        ],
    )(dp, p)


def _tc_matmul_body(dp_ref, h0_ref, h1_ref, p_ref, w0_ref, w1_ref, w2_ref,
                    b_ref, out_ref, g_ref):
    norm = _norm_block(dp_ref[...])
    h2 = (p_ref[0] + p_ref[1]) * norm
    acc = jnp.dot(h0_ref[...], w0_ref[...], preferred_element_type=jnp.float32)
    acc += jnp.dot(h1_ref[...], w1_ref[...], preferred_element_type=jnp.float32)
    acc += jnp.dot(h2, w2_ref[...], preferred_element_type=jnp.float32)
    out = jnp.maximum(acc + b_ref[...], 0.0)
    out_ref[...] = out
    g_ref[...] = out * norm


def _tc_matmul(dp, h0, h1, p, w0, w1, w2, b):
    return pl.pallas_call(
        _tc_matmul_body,
        grid=(_NP // _R,),
        in_specs=[
            pl.BlockSpec((_NC, _R, 16), lambda i: (0, i, 0)),
            pl.BlockSpec((_R, _D), lambda i: (i, 0)),
            pl.BlockSpec((_R, _D), lambda i: (i, 0)),
            pl.BlockSpec((_NC, _R, _D), lambda i: (0, i, 0)),
            pl.BlockSpec((_D, _D), lambda i: (0, 0)),
            pl.BlockSpec((_D, _D), lambda i: (0, 0)),
            pl.BlockSpec((_D, _D), lambda i: (0, 0)),
            pl.BlockSpec((1, _D), lambda i: (0, 0)),
        ],
        out_specs=[
            pl.BlockSpec((_R, _D), lambda i: (i, 0)),
            pl.BlockSpec((_R, _D), lambda i: (i, 0)),
        ],
        out_shape=[
            jax.ShapeDtypeStruct((_NP, _D), jnp.float32),
            jax.ShapeDtypeStruct((_NP, _D), jnp.float32),
        ],
    )(dp, h0, h1, p, w0, w1, w2, b)


def kernel(x, edge_index, W0, b0, W1, b1):
    src = edge_index[0].astype(jnp.int32)
    dst = edge_index[1].astype(jnp.int32)
    pad = _N + (jnp.arange(_EP - _E, dtype=jnp.int32) % _NPAD)
    src_s = jnp.concatenate([src, pad]).reshape(_NW, _CPW, _CH)
    dst_s = jnp.concatenate([dst, pad]).reshape(_NW, _CPW, _CH)
    xp = jnp.pad(x, ((0, _NP - _N), (0, 0)))
    b0r = b0.reshape(1, _D)
    b1r = b1.reshape(1, _D)

    dp = _sc_degree(dst_s)

    g0 = _tc_scale(dp, xp)
    p1 = _sc_propagate(g0, src_s, dst_s)
    h1, g1 = _tc_combine(dp, p1)
    p2 = _sc_propagate(g1, src_s, dst_s)
    hL1, gL1 = _tc_matmul(dp, xp, h1, p2,
                          W0[0:_D], W0[_D:2 * _D], W0[2 * _D:3 * _D], b0r)

    p3 = _sc_propagate(gL1, src_s, dst_s)
    h1b, g1b = _tc_combine(dp, p3)
    p4 = _sc_propagate(g1b, src_s, dst_s)
    out, _g = _tc_matmul(dp, hL1, h1b, p4,
                         W1[0:_D], W1[_D:2 * _D], W1[2 * _D:3 * _D], b1r)

    return out[:_N]
